# baseline (speedup 1.0000x reference)
import functools

import jax
import jax.numpy as jnp
from jax import lax
from jax.experimental import pallas as pl
from jax.experimental.pallas import tpu as pltpu
from jax.experimental.pallas import tpu_sc as plsc

B = 16
N = 32768
D = 1280
H = 128

ROWS = 2048
N_SC = 10240
N_TC = N - N_SC
TC_BLK0 = N_SC // ROWS
NBLK = N_TC // ROWS
IDS_R = ROWS // 128

NW = 32
RPW = N_SC // NW
C = 32
NCHUNK = RPW // C
LANES = 16
NCOL = D // LANES


def _sc_body(x_hbm, ids_hbm, sums_hbm, cnt_hbm,
             ids_v, acc_v, cnt_v, buf0, buf1, sem0, sem1):
    wid = lax.axis_index("s") * 2 + lax.axis_index("c")
    base = wid * RPW

    pltpu.async_copy(x_hbm.at[pl.ds(base, C)], buf0, sem0)
    pltpu.async_copy(x_hbm.at[pl.ds(base + C, C)], buf1, sem1)

    pltpu.sync_copy(ids_hbm.at[pl.ds(base, RPW)], ids_v.at[pl.ds(0, RPW)])

    iota16 = lax.iota(jnp.int32, LANES)
    zeros16 = jnp.zeros((LANES,), jnp.float32)

    @plsc.parallel_loop(0, NCOL, unroll=1)
    def _zbody(i):
        col = i * LANES
        for s in range(B):
            acc_v[s, pl.ds(col, LANES)] = zeros16
    cnt_v[...] = zeros16

    def treesum(vals):
        while len(vals) > 1:
            nxt = [vals[i] + vals[i + 1] for i in range(0, len(vals) - 1, 2)]
            if len(vals) % 2:
                nxt.append(vals[-1])
            vals = nxt
        return vals[0]

    bufs = (buf0, buf1)
    sems = (sem0, sem1)

    def process(buf, off):
        v0 = ids_v[pl.ds(off, LANES)]
        v1 = ids_v[pl.ds(off + LANES, LANES)]
        s0 = v0[0]
        s_splat = jnp.broadcast_to(s0, (LANES,))
        uniform = jnp.all((v0 == s_splat) & (v1 == s_splat))

        def hot():
            @plsc.parallel_loop(0, NCOL, unroll=1)
            def _jbody(j):
                col = j * LANES
                colsum = treesum(
                    [buf[r, pl.ds(col, LANES)] for r in range(C)])
                plsc.addupdate(acc_v.at[s0, pl.ds(col, LANES)], colsum)
            cnt_v[...] += jnp.where(iota16 == s_splat, float(C), 0.0)

        def cold():
            def rbody(r, _):
                s = ids_v[pl.ds(off + r, LANES)][0]

                @plsc.parallel_loop(0, NCOL, unroll=1)
                def _jbody(j):
                    col = j * LANES
                    plsc.addupdate(acc_v.at[s, pl.ds(col, LANES)],
                                   buf[r, pl.ds(col, LANES)])
                cnt_v[...] += jnp.where(
                    iota16 == jnp.broadcast_to(s, (LANES,)), 1.0, 0.0)
                return 0
            lax.fori_loop(0, C, rbody, 0)

        lax.cond(uniform, hot, cold)

    def pair_body(k2, _):
        for b in range(2):
            k = k2 * 2 + b
            pltpu.make_async_copy(
                x_hbm.at[pl.ds(base + k * C, C)], bufs[b], sems[b]).wait()
            process(bufs[b], k * C)

            @pl.when(k + 2 < NCHUNK)
            def _next():
                pltpu.async_copy(
                    x_hbm.at[pl.ds(base + (k + 2) * C, C)], bufs[b], sems[b])
        return 0
    lax.fori_loop(0, NCHUNK // 2, pair_body, 0)

    pltpu.sync_copy(acc_v, sums_hbm.at[pl.ds(wid * B, B)])
    pltpu.sync_copy(cnt_v, cnt_hbm.at[wid])


def _sc_partials(prot_x, ids32):
    mesh = plsc.VectorSubcoreMesh(core_axis_name="c", subcore_axis_name="s")
    f = pl.kernel(
        _sc_body,
        mesh=mesh,
        compiler_params=pltpu.CompilerParams(needs_layout_passes=False),
        out_type=[
            jax.ShapeDtypeStruct((NW * B, D), jnp.float32),
            jax.ShapeDtypeStruct((NW, LANES), jnp.float32),
        ],
        scratch_types=[
            pltpu.VMEM((RPW + LANES,), jnp.int32),
            pltpu.VMEM((B, D), jnp.float32),
            pltpu.VMEM((LANES,), jnp.float32),
            pltpu.VMEM((C, D), jnp.float32),
            pltpu.VMEM((C, D), jnp.float32),
            pltpu.SemaphoreType.DMA,
            pltpu.SemaphoreType.DMA,
        ],
    )
    return f(prot_x, ids32)


def _tc_seg_kernel(x_ref, ids_ref, sum_ref, cnt_ref):
    i = pl.program_id(0)

    @pl.when(i == 0)
    def _init():
        sum_ref[...] = jnp.zeros_like(sum_ref)
        cnt_ref[...] = jnp.zeros_like(cnt_ref)

    seg = ids_ref[...].reshape(1, ROWS)
    bidx = lax.broadcasted_iota(jnp.int32, (B, ROWS), 0)
    onehot = (bidx == seg).astype(jnp.float32)
    sum_ref[...] += jnp.dot(onehot, x_ref[...],
                            preferred_element_type=jnp.float32)
    cnt_ref[...] += jnp.sum(onehot, axis=1, keepdims=True)


def _tc_partials(prot_x, ids2d):
    return pl.pallas_call(
        _tc_seg_kernel,
        grid=(NBLK,),
        in_specs=[
            pl.BlockSpec((ROWS, D), lambda i: (i + TC_BLK0, 0)),
            pl.BlockSpec((IDS_R, 128), lambda i: (i + TC_BLK0, 0)),
        ],
        out_specs=[
            pl.BlockSpec((B, D), lambda i: (0, 0)),
            pl.BlockSpec((B, 1), lambda i: (0, 0)),
        ],
        out_shape=[
            jax.ShapeDtypeStruct((B, D), jnp.float32),
            jax.ShapeDtypeStruct((B, 1), jnp.float32),
        ],
    )(prot_x, ids2d)


def _head_kernel(scs_ref, scc_ref, tcs_ref, tcc_ref, mol_ref,
                 wp_ref, bp_ref, wf_ref, bf_ref, out_ref, pgf_ref):
    total = tcs_ref[...]
    for k in range(NW):
        total = total + scs_ref[k * B:(k + 1) * B, :]
    cnt_row = jnp.sum(scc_ref[...], axis=0, keepdims=True)
    eye = (lax.broadcasted_iota(jnp.int32, (B, B), 0)
           == lax.broadcasted_iota(jnp.int32, (B, B), 1)).astype(jnp.float32)
    cnt_col = jnp.sum(eye * cnt_row, axis=1, keepdims=True)
    cnt_col = cnt_col + tcc_ref[...]
    mean = total / jnp.maximum(cnt_col, 1.0)
    nonempty = (cnt_col > 0.0).astype(jnp.float32)
    pgf = (jnp.dot(mean, wp_ref[...], preferred_element_type=jnp.float32)
           + bp_ref[...] * nonempty)
    pgf_ref[...] = pgf
    w_mol = wf_ref[:, :H]
    w_pgf = wf_ref[:, H:]
    out_ref[...] = (jnp.sum(mol_ref[...] * w_mol, axis=1, keepdims=True)
                    + jnp.sum(pgf * w_pgf, axis=1, keepdims=True)
                    + bf_ref[0, 0])


@jax.jit
def _run(prot_x, mol_feat, ids32, ids2d, W_prot, b_prot2d, W_ffn_t, b_ffn2d):
    sc_sums, sc_cnts = _sc_partials(prot_x, ids32)
    tc_sums, tc_cnts = _tc_partials(prot_x, ids2d)
    out, pgf = pl.pallas_call(
        _head_kernel,
        out_shape=[
            jax.ShapeDtypeStruct((B, 1), jnp.float32),
            jax.ShapeDtypeStruct((B, H), jnp.float32),
        ],
    )(sc_sums, sc_cnts, tc_sums, tc_cnts, mol_feat,
      W_prot, b_prot2d, W_ffn_t, b_ffn2d)
    return out, pgf


def kernel(prot_x, mol_feat, segment_ids, W_prot, b_prot, W_ffn, b_ffn):
    ids32 = segment_ids.astype(jnp.int32)
    out, pgf = _run(prot_x, mol_feat, ids32, ids32.reshape(N // 128, 128),
                    W_prot, b_prot.reshape(1, H),
                    W_ffn.reshape(1, 2 * H).astype(jnp.float32),
                    b_ffn.reshape(1, 1))
    return (out, mol_feat, pgf)


# speedup vs baseline: 1.0014x; 1.0014x over previous
import functools

import jax
import jax.numpy as jnp
from jax import lax
from jax.experimental import pallas as pl
from jax.experimental.pallas import tpu as pltpu
from jax.experimental.pallas import tpu_sc as plsc

B = 16
N = 32768
D = 1280
H = 128

ROWS = 2048
N_SC = 10240
N_TC = N - N_SC
TC_BLK0 = N_SC // ROWS
NBLK = N_TC // ROWS
IDS_R = ROWS // 128

NW = 32
RPW = N_SC // NW
C = 32
NCHUNK = RPW // C
LANES = 16
NCOL = D // LANES


def _sc_body(x_hbm, ids_hbm, sums_hbm, cnt_hbm,
             ids_v, acc_v, cnt_v, buf0, buf1, sem0, sem1):
    wid = lax.axis_index("s") * 2 + lax.axis_index("c")
    base = wid * RPW

    pltpu.async_copy(x_hbm.at[pl.ds(base, C)], buf0, sem0)
    pltpu.async_copy(x_hbm.at[pl.ds(base + C, C)], buf1, sem1)

    pltpu.sync_copy(ids_hbm.at[pl.ds(base, RPW)], ids_v.at[pl.ds(0, RPW)])

    iota16 = lax.iota(jnp.int32, LANES)
    zeros16 = jnp.zeros((LANES,), jnp.float32)

    @plsc.parallel_loop(0, NCOL, unroll=1)
    def _zbody(i):
        col = i * LANES
        for s in range(B):
            acc_v[s, pl.ds(col, LANES)] = zeros16
    cnt_v[...] = zeros16

    def treesum(vals):
        while len(vals) > 1:
            nxt = [vals[i] + vals[i + 1] for i in range(0, len(vals) - 1, 2)]
            if len(vals) % 2:
                nxt.append(vals[-1])
            vals = nxt
        return vals[0]

    bufs = (buf0, buf1)
    sems = (sem0, sem1)

    def process(buf, off):
        v0 = ids_v[pl.ds(off, LANES)]
        v1 = ids_v[pl.ds(off + LANES, LANES)]
        s0 = v0[0]
        s_splat = jnp.broadcast_to(s0, (LANES,))
        uniform = jnp.all((v0 == s_splat) & (v1 == s_splat))

        def hot():
            @plsc.parallel_loop(0, NCOL, unroll=1)
            def _jbody(j):
                col = j * LANES
                colsum = treesum(
                    [buf[r, pl.ds(col, LANES)] for r in range(C)])
                plsc.addupdate(acc_v.at[s0, pl.ds(col, LANES)], colsum)
            cnt_v[...] += jnp.where(iota16 == s_splat, float(C), 0.0)

        def cold():
            def rbody(r, _):
                s = ids_v[pl.ds(off + r, LANES)][0]

                @plsc.parallel_loop(0, NCOL, unroll=1)
                def _jbody(j):
                    col = j * LANES
                    plsc.addupdate(acc_v.at[s, pl.ds(col, LANES)],
                                   buf[r, pl.ds(col, LANES)])
                cnt_v[...] += jnp.where(
                    iota16 == jnp.broadcast_to(s, (LANES,)), 1.0, 0.0)
                return 0
            lax.fori_loop(0, C, rbody, 0)

        lax.cond(uniform, hot, cold)

    def pair_body(k2, _):
        for b in range(2):
            k = k2 * 2 + b
            pltpu.make_async_copy(
                x_hbm.at[pl.ds(base + k * C, C)], bufs[b], sems[b]).wait()
            process(bufs[b], k * C)

            @pl.when(k + 2 < NCHUNK)
            def _next():
                pltpu.async_copy(
                    x_hbm.at[pl.ds(base + (k + 2) * C, C)], bufs[b], sems[b])
        return 0
    lax.fori_loop(0, NCHUNK // 2, pair_body, 0)

    pltpu.sync_copy(acc_v, sums_hbm.at[pl.ds(wid * B, B)])
    pltpu.sync_copy(cnt_v, cnt_hbm.at[wid])


def _sc_partials(prot_x, ids32):
    mesh = plsc.VectorSubcoreMesh(core_axis_name="c", subcore_axis_name="s")
    f = pl.kernel(
        _sc_body,
        mesh=mesh,
        compiler_params=pltpu.CompilerParams(needs_layout_passes=False,
                                             skip_device_barrier=True),
        out_type=[
            jax.ShapeDtypeStruct((NW * B, D), jnp.float32),
            jax.ShapeDtypeStruct((NW, LANES), jnp.float32),
        ],
        scratch_types=[
            pltpu.VMEM((RPW + LANES,), jnp.int32),
            pltpu.VMEM((B, D), jnp.float32),
            pltpu.VMEM((LANES,), jnp.float32),
            pltpu.VMEM((C, D), jnp.float32),
            pltpu.VMEM((C, D), jnp.float32),
            pltpu.SemaphoreType.DMA,
            pltpu.SemaphoreType.DMA,
        ],
    )
    return f(prot_x, ids32)


def _tc_seg_kernel(x_ref, ids_ref, sum_ref, cnt_ref):
    i = pl.program_id(0)

    @pl.when(i == 0)
    def _init():
        sum_ref[...] = jnp.zeros_like(sum_ref)
        cnt_ref[...] = jnp.zeros_like(cnt_ref)

    seg = ids_ref[...].reshape(1, ROWS)
    bidx = lax.broadcasted_iota(jnp.int32, (B, ROWS), 0)
    onehot = (bidx == seg).astype(jnp.float32)
    sum_ref[...] += jnp.dot(onehot, x_ref[...],
                            preferred_element_type=jnp.float32)
    cnt_ref[...] += jnp.sum(onehot, axis=1, keepdims=True)


def _tc_partials(prot_x, ids2d):
    return pl.pallas_call(
        _tc_seg_kernel,
        grid=(NBLK,),
        in_specs=[
            pl.BlockSpec((ROWS, D), lambda i: (i + TC_BLK0, 0)),
            pl.BlockSpec((IDS_R, 128), lambda i: (i + TC_BLK0, 0)),
        ],
        out_specs=[
            pl.BlockSpec((B, D), lambda i: (0, 0)),
            pl.BlockSpec((B, 1), lambda i: (0, 0)),
        ],
        out_shape=[
            jax.ShapeDtypeStruct((B, D), jnp.float32),
            jax.ShapeDtypeStruct((B, 1), jnp.float32),
        ],
    )(prot_x, ids2d)


def _head_kernel(scs_ref, scc_ref, tcs_ref, tcc_ref, mol_ref,
                 wp_ref, bp_ref, wf_ref, bf_ref, out_ref, pgf_ref):
    total = tcs_ref[...]
    for k in range(NW):
        total = total + scs_ref[k * B:(k + 1) * B, :]
    cnt_row = jnp.sum(scc_ref[...], axis=0, keepdims=True)
    eye = (lax.broadcasted_iota(jnp.int32, (B, B), 0)
           == lax.broadcasted_iota(jnp.int32, (B, B), 1)).astype(jnp.float32)
    cnt_col = jnp.sum(eye * cnt_row, axis=1, keepdims=True)
    cnt_col = cnt_col + tcc_ref[...]
    mean = total / jnp.maximum(cnt_col, 1.0)
    nonempty = (cnt_col > 0.0).astype(jnp.float32)
    pgf = (jnp.dot(mean, wp_ref[...], preferred_element_type=jnp.float32)
           + bp_ref[...] * nonempty)
    pgf_ref[...] = pgf
    w_mol = wf_ref[:, :H]
    w_pgf = wf_ref[:, H:]
    out_ref[...] = (jnp.sum(mol_ref[...] * w_mol, axis=1, keepdims=True)
                    + jnp.sum(pgf * w_pgf, axis=1, keepdims=True)
                    + bf_ref[0, 0])


@jax.jit
def _run(prot_x, mol_feat, ids32, ids2d, W_prot, b_prot2d, W_ffn_t, b_ffn2d):
    sc_sums, sc_cnts = _sc_partials(prot_x, ids32)
    tc_sums, tc_cnts = _tc_partials(prot_x, ids2d)
    out, pgf = pl.pallas_call(
        _head_kernel,
        out_shape=[
            jax.ShapeDtypeStruct((B, 1), jnp.float32),
            jax.ShapeDtypeStruct((B, H), jnp.float32),
        ],
    )(sc_sums, sc_cnts, tc_sums, tc_cnts, mol_feat,
      W_prot, b_prot2d, W_ffn_t, b_ffn2d)
    return out, pgf


def kernel(prot_x, mol_feat, segment_ids, W_prot, b_prot, W_ffn, b_ffn):
    ids32 = segment_ids.astype(jnp.int32)
    out, pgf = _run(prot_x, mol_feat, ids32, ids32.reshape(N // 128, 128),
                    W_prot, b_prot.reshape(1, H),
                    W_ffn.reshape(1, 2 * H).astype(jnp.float32),
                    b_ffn.reshape(1, 1))
    return (out, mol_feat, pgf)
